# baseline (speedup 1.0000x reference)
import functools

import jax
import jax.numpy as jnp
from jax import lax
from jax.experimental import pallas as pl
from jax.experimental.pallas import tpu as pltpu
from jax.experimental.pallas import tpu_sc as plsc

S = 2048
HD = 64
T = 2 * S + 1
ROW = S * HD
NC = 2
NS = 16
NW = NC * NS
RPW = S // NW
HALF = S // 2
WIN = (HALF + RPW) * HD
CHUNK = HALF * HD

_mesh = plsc.VectorSubcoreMesh(core_axis_name="c", subcore_axis_name="s")


@functools.partial(
    pl.kernel,
    mesh=_mesh,
    out_type=jax.ShapeDtypeStruct((S * ROW,), jnp.float32),
    scratch_types=[
        pltpu.VMEM((WIN,), jnp.float32),
        pltpu.SemaphoreType.DMA,
    ],
)
def _relpos_rows(table_hbm, out_hbm, win, sem):
    c = lax.axis_index("c")
    s = lax.axis_index("s")
    wid = s * NC + c
    i0 = wid * RPW

    def half(h, carry):
        w0 = (S - i0 - (RPW - 1) + h * HALF) * HD
        pltpu.sync_copy(table_hbm.at[pl.ds(w0, WIN)], win)

        def row_start(r, inner):
            i = i0 + r
            src = (RPW - 1 - r) * HD
            dst = i * ROW + h * CHUNK
            pltpu.async_copy(win.at[pl.ds(src, CHUNK)], out_hbm.at[pl.ds(dst, CHUNK)], sem)
            return inner

        lax.fori_loop(0, RPW, row_start, 0)

        def row_wait(r, inner):
            pltpu.make_async_copy(
                win.at[pl.ds(0, CHUNK)], out_hbm.at[pl.ds(0, CHUNK)], sem
            ).wait()
            return inner

        return lax.fori_loop(0, RPW, row_wait, carry)

    lax.fori_loop(0, 2, half, 0)


def kernel(seq_length, table):
    del seq_length
    flat = table.reshape(T * HD)
    out = _relpos_rows(flat)
    return out.reshape(S, S, HD)


# speedup vs baseline: 4.5064x; 4.5064x over previous
import functools

import jax
import jax.numpy as jnp
from jax import lax
from jax.experimental import pallas as pl
from jax.experimental.pallas import tpu as pltpu
from jax.experimental.pallas import tpu_sc as plsc

S = 2048
HD = 64
TR = 4096
NW = 32
GROUP_ROWS = 8 * TR

_mesh = plsc.VectorSubcoreMesh(core_axis_name="c", subcore_axis_name="s")


@functools.partial(
    pl.kernel,
    mesh=_mesh,
    out_type=jax.ShapeDtypeStruct((S, 8, 16, 8, 128), jnp.float32),
    scratch_types=[
        pltpu.VMEM((GROUP_ROWS + 128,), jnp.float32),
        pltpu.VMEM((32, 8, 128), jnp.float32),
        pltpu.SemaphoreType.DMA,
    ],
)
def _relpos_slabs(t3_hbm, out_hbm, buf, tst, sem):
    c = lax.axis_index("c")
    s = lax.axis_index("s")
    wid = s * 2 + c
    g = wid // 4
    o_base = (wid % 4) * 32

    pltpu.sync_copy(t3_hbm.at[pl.ds(g * GROUP_ROWS, GROUP_ROWS)], buf.at[pl.ds(0, GROUP_ROWS)])

    def otask(oo, carry):
        o = o_base + oo

        def build(m, inner):
            for hd8 in range(8):
                for k in range(8):
                    v = buf[pl.ds(hd8 * TR + 128 * m + o + 16 * k, 16)]
                    tst[m, hd8, pl.ds(16 * k, 16)] = v
            return inner

        lax.fori_loop(0, 32, build, 0)

        def fire(m0, inner):
            i = S - o - 128 * m0

            @pl.when(jnp.logical_and(i >= 0, i < S))
            def _():
                pltpu.async_copy(tst.at[pl.ds(m0, 16)], out_hbm.at[i, g], sem)

            return inner

        lax.fori_loop(0, 17, fire, 0)

        def drain(r, inner):
            pltpu.make_async_copy(
                tst.at[pl.ds(0, 16)], out_hbm.at[0, 0], sem
            ).wait()
            return inner

        return lax.fori_loop(0, 16, drain, carry)

    lax.fori_loop(0, 32, otask, 0)


def kernel(seq_length, table):
    del seq_length
    t3 = jnp.transpose(table[:TR]).reshape(HD * TR)
    out5 = _relpos_slabs(t3)
    return jnp.transpose(out5, (0, 2, 4, 1, 3)).reshape(S, S, HD)


# speedup vs baseline: 7.5223x; 1.6693x over previous
import functools

import jax
import jax.numpy as jnp
from jax import lax
from jax.experimental import pallas as pl
from jax.experimental.pallas import tpu as pltpu
from jax.experimental.pallas import tpu_sc as plsc

S = 2048
HD = 64
TR = 4096
NW = 32
GROUP_ROWS = 8 * TR

_mesh = plsc.VectorSubcoreMesh(core_axis_name="c", subcore_axis_name="s")


@functools.partial(
    pl.kernel,
    mesh=_mesh,
    out_type=jax.ShapeDtypeStruct((S, 8, 16, 8, 128), jnp.float32),
    scratch_types=[
        pltpu.VMEM((GROUP_ROWS + 128,), jnp.float32),
        pltpu.VMEM((2, 32, 8, 128), jnp.float32),
        pltpu.SemaphoreType.DMA,
    ],
)
def _relpos_slabs(t3_hbm, out_hbm, buf, tst, sem):
    c = lax.axis_index("c")
    s = lax.axis_index("s")
    wid = s * 2 + c
    g = wid // 4
    o_base = (wid % 4) * 32

    pltpu.sync_copy(t3_hbm.at[pl.ds(g * GROUP_ROWS, GROUP_ROWS)], buf.at[pl.ds(0, GROUP_ROWS)])

    def drain(r, inner):
        pltpu.make_async_copy(
            tst.at[0, pl.ds(0, 16)], out_hbm.at[0, 0], sem
        ).wait()
        return inner

    def otask(oo, carry):
        o = o_base + oo
        p = oo % 2

        @pl.when(oo >= 2)
        def _():
            lax.fori_loop(0, 16, drain, 0)

        def build(m, inner):
            for hd8 in range(8):
                for k in range(8):
                    v = buf[pl.ds(hd8 * TR + 128 * m + o + 16 * k, 16)]
                    tst[p, m, hd8, pl.ds(16 * k, 16)] = v
            return inner

        lax.fori_loop(0, 32, build, 0)

        def fire(m0, inner):
            i = S - o - 128 * m0

            @pl.when(jnp.logical_and(i >= 0, i < S))
            def _():
                pltpu.async_copy(tst.at[p, pl.ds(m0, 16)], out_hbm.at[i, g], sem)

            return inner

        return lax.fori_loop(0, 17, fire, carry)

    lax.fori_loop(0, 32, otask, 0)
    lax.fori_loop(0, 32, drain, 0)


def kernel(seq_length, table):
    del seq_length
    t3 = jnp.transpose(table[:TR]).reshape(HD * TR)
    out5 = _relpos_slabs(t3)
    return jnp.transpose(out5, (0, 2, 4, 1, 3)).reshape(S, S, HD)
